# baseline (speedup 1.0000x reference)
import functools

import numpy as np
import jax
import jax.numpy as jnp
from jax import lax
from jax.experimental import pallas as pl
from jax.experimental.pallas import tpu as pltpu
from jax.experimental.pallas import tpu_sc as plsc

_F = 26
_D = 32
_NW = 32
_CHUNK = 128


def _sc_gather(tables, idx_t):
    nw, f, _ = idx_t.shape
    d = tables.shape[2]
    mesh = plsc.VectorSubcoreMesh(core_axis_name="c", subcore_axis_name="s")

    @functools.partial(
        pl.kernel,
        mesh=mesh,
        compiler_params=pltpu.CompilerParams(use_tc_tiling_on_sc=False),
        out_type=jax.ShapeDtypeStruct((nw * 128, f, d), jnp.float32),
        scratch_types=[
            pltpu.VMEM((f, _CHUNK), jnp.int32),
            pltpu.VMEM((f * _CHUNK, d), jnp.float32),
            pltpu.SemaphoreType.DMA,
        ],
    )
    def gather_kernel(tab_hbm, idx_hbm, out_hbm, idx_v, rows_v, sem):
        wid = lax.axis_index("s") * 2 + lax.axis_index("c")
        pltpu.sync_copy(idx_hbm.at[wid], idx_v)
        copies = []
        for j in range(f):
            copies.append(
                pltpu.async_copy(
                    tab_hbm.at[j].at[idx_v.at[j]],
                    rows_v.at[pl.ds(j * _CHUNK, _CHUNK)],
                    sem,
                )
            )
        for cp in copies:
            cp.wait()
        for j in range(f):
            pltpu.sync_copy(
                rows_v.at[pl.ds(j * _CHUNK, _CHUNK)],
                out_hbm.at[pl.ds(wid * _CHUNK, _CHUNK), j])

    return gather_kernel(tables, idx_t)


def _tc_forward_body(x_ref, e_ref, w0_ref, b0_ref, w1_ref, b1_ref, w2_ref,
                     b2_ref, wt_ref, bt_ref, seg_ref, wz_ref, out_ref):
    f32 = jnp.float32
    x = x_ref[...]
    h = jnp.maximum(jnp.dot(x, w0_ref[...], preferred_element_type=f32)
                    + b0_ref[...], 0.0)
    h = jnp.maximum(jnp.dot(h, w1_ref[...], preferred_element_type=f32)
                    + b1_ref[...], 0.0)
    h = jnp.maximum(jnp.dot(h, w2_ref[...], preferred_element_type=f32)
                    + b2_ref[...], 0.0)

    e2d = e_ref[...]
    seg = seg_ref[...]
    dots = []
    for i in range(_F):
        a = e2d[:, i * _D : (i + 1) * _D]
        tiled = pltpu.repeat(a, _F, axis=1)
        dots.append(jnp.dot(e2d * tiled, seg, preferred_element_type=f32))
    dcat = jnp.concatenate(dots, axis=1)

    out = (jnp.dot(h, wt_ref[...], preferred_element_type=f32)
           + jnp.dot(dcat, wz_ref[...], preferred_element_type=f32)
           + bt_ref[...])
    out_ref[...] = out


def _tc_forward(dense, embeds2d, w0, b0, w1, b1, w2, b2, wt_top, bt, seg, wz,
                interpret=False):
    batch, fd = embeds2d.shape
    bblk = 512
    grid = (batch // bblk,)
    top_out = wt_top.shape[1]

    def full(shape):
        return pl.BlockSpec(shape, lambda i: (0,) * len(shape))

    return pl.pallas_call(
        _tc_forward_body,
        grid=grid,
        in_specs=[
            pl.BlockSpec((bblk, dense.shape[1]), lambda i: (i, 0)),
            pl.BlockSpec((bblk, fd), lambda i: (i, 0)),
            full(w0.shape), full(b0.shape),
            full(w1.shape), full(b1.shape),
            full(w2.shape), full(b2.shape),
            full(wt_top.shape), full(bt.shape),
            full(seg.shape), full(wz.shape),
        ],
        out_specs=pl.BlockSpec((bblk, top_out), lambda i: (i, 0)),
        out_shape=jax.ShapeDtypeStruct((batch, top_out), jnp.float32),
        interpret=interpret,
    )(dense, embeds2d, w0, b0, w1, b1, w2, b2, wt_top, bt, seg, wz)


def kernel(dense_features, sparse_features, W0, b0, W1, b1, W2, b2, tables,
           Wt, bt):
    f, v, d = tables.shape
    batch = dense_features.shape[0]
    n_mlp_out = W2.shape[1]

    idx_t = jnp.swapaxes(
        sparse_features.astype(jnp.int32).reshape(_NW, _CHUNK, f), 1, 2)

    embeds2d = _sc_gather(tables, idx_t).reshape(batch, f * d)

    seg = jnp.asarray(
        np.repeat(np.eye(f, dtype=np.float32), d, axis=0))

    iu0, iu1 = np.triu_indices(f, k=1)
    flat_pos = jnp.asarray(iu0 * f + iu1, dtype=jnp.int32)
    wz = jnp.zeros((f * f, Wt.shape[1]), jnp.float32).at[flat_pos].set(
        Wt[n_mlp_out:])

    return _tc_forward(dense_features, embeds2d, W0, b0[None, :], W1,
                       b1[None, :], W2, b2[None, :], Wt[:n_mlp_out],
                       bt[None, :], seg, wz)


# speedup vs baseline: 1.0504x; 1.0504x over previous
import functools

import numpy as np
import jax
import jax.numpy as jnp
from jax import lax
from jax.experimental import pallas as pl
from jax.experimental.pallas import tpu as pltpu
from jax.experimental.pallas import tpu_sc as plsc

_F = 26
_D = 32
_NW = 32
_CHUNK = 128


def _sc_gather(flat_tables, idx3):
    nw, c, _ = idx3.shape
    rows_per_w = c * _CHUNK
    total = nw * rows_per_w
    d = flat_tables.shape[1]
    mesh = plsc.VectorSubcoreMesh(core_axis_name="c", subcore_axis_name="s")

    @functools.partial(
        pl.kernel,
        mesh=mesh,
        compiler_params=pltpu.CompilerParams(use_tc_tiling_on_sc=False),
        out_type=jax.ShapeDtypeStruct((total, d), jnp.float32),
        scratch_types=[
            pltpu.VMEM((c, _CHUNK), jnp.int32),
            pltpu.VMEM((rows_per_w, d), jnp.float32),
            pltpu.SemaphoreType.DMA,
        ],
    )
    def gather_kernel(tab_hbm, idx_hbm, out_hbm, idx_v, rows_v, sem):
        wid = lax.axis_index("s") * 2 + lax.axis_index("c")
        pltpu.sync_copy(idx_hbm.at[wid], idx_v)
        copies = []
        for j in range(c):
            copies.append(
                pltpu.async_copy(
                    tab_hbm.at[idx_v.at[j]],
                    rows_v.at[pl.ds(j * _CHUNK, _CHUNK)],
                    sem,
                )
            )
        for cp in copies:
            cp.wait()
        pltpu.sync_copy(rows_v, out_hbm.at[pl.ds(wid * rows_per_w, rows_per_w)])

    return gather_kernel(flat_tables, idx3)


def _tc_forward_body(x_ref, e_ref, w0_ref, b0_ref, w1_ref, b1_ref, w2_ref,
                     b2_ref, wt_ref, bt_ref, seg_ref, wz_ref, out_ref):
    f32 = jnp.float32
    x = x_ref[...]
    h = jnp.maximum(jnp.dot(x, w0_ref[...], preferred_element_type=f32)
                    + b0_ref[...], 0.0)
    h = jnp.maximum(jnp.dot(h, w1_ref[...], preferred_element_type=f32)
                    + b1_ref[...], 0.0)
    h = jnp.maximum(jnp.dot(h, w2_ref[...], preferred_element_type=f32)
                    + b2_ref[...], 0.0)

    e2d = e_ref[...]
    seg = seg_ref[...]
    dots = []
    for i in range(_F):
        a = e2d[:, i * _D : (i + 1) * _D]
        tiled = pltpu.repeat(a, _F, axis=1)
        dots.append(jnp.dot(e2d * tiled, seg, preferred_element_type=f32))
    dcat = jnp.concatenate(dots, axis=1)

    out = (jnp.dot(h, wt_ref[...], preferred_element_type=f32)
           + jnp.dot(dcat, wz_ref[...], preferred_element_type=f32)
           + bt_ref[...])
    out_ref[...] = out


def _tc_forward(dense, embeds2d, w0, b0, w1, b1, w2, b2, wt_top, bt, seg, wz,
                interpret=False):
    batch, fd = embeds2d.shape
    bblk = 512
    grid = (batch // bblk,)
    top_out = wt_top.shape[1]

    def full(shape):
        return pl.BlockSpec(shape, lambda i: (0,) * len(shape))

    return pl.pallas_call(
        _tc_forward_body,
        grid=grid,
        in_specs=[
            pl.BlockSpec((bblk, dense.shape[1]), lambda i: (i, 0)),
            pl.BlockSpec((bblk, fd), lambda i: (i, 0)),
            full(w0.shape), full(b0.shape),
            full(w1.shape), full(b1.shape),
            full(w2.shape), full(b2.shape),
            full(wt_top.shape), full(bt.shape),
            full(seg.shape), full(wz.shape),
        ],
        out_specs=pl.BlockSpec((bblk, top_out), lambda i: (i, 0)),
        out_shape=jax.ShapeDtypeStruct((batch, top_out), jnp.float32),
        interpret=interpret,
    )(dense, embeds2d, w0, b0, w1, b1, w2, b2, wt_top, bt, seg, wz)


def kernel(dense_features, sparse_features, W0, b0, W1, b1, W2, b2, tables,
           Wt, bt):
    f, v, d = tables.shape
    batch = dense_features.shape[0]
    n_mlp_out = W2.shape[1]

    offs = (jnp.arange(f, dtype=jnp.int32) * v)[None, :]
    idx = (sparse_features.astype(jnp.int32) + offs).reshape(-1)
    idx3 = idx.reshape(_NW, (batch * f) // (_NW * _CHUNK), _CHUNK)

    flat_tables = tables.reshape(f * v, d)
    embeds2d = _sc_gather(flat_tables, idx3).reshape(batch, f * d)

    seg = jnp.asarray(
        np.repeat(np.eye(f, dtype=np.float32), d, axis=0))

    iu0, iu1 = np.triu_indices(f, k=1)
    flat_pos = jnp.asarray(iu0 * f + iu1, dtype=jnp.int32)
    wz = jnp.zeros((f * f, Wt.shape[1]), jnp.float32).at[flat_pos].set(
        Wt[n_mlp_out:])

    return _tc_forward(dense_features, embeds2d, W0, b0[None, :], W1,
                       b1[None, :], W2, b2[None, :], Wt[:n_mlp_out],
                       bt[None, :], seg, wz)
